# baseline (speedup 1.0000x reference)
import functools

import jax
import jax.numpy as jnp
from jax import lax
from jax.experimental import pallas as pl
from jax.experimental.pallas import tpu as pltpu
from jax.experimental.pallas import tpu_sc as plsc

NC = 2
NS = 16
NW = NC * NS


def _mm3_body(h_ref, w_ref, b_ref, s_ref, bt_ref, c_ref):
  acc = jnp.dot(h_ref[...], w_ref[...], preferred_element_type=jnp.float32)
  acc = acc + b_ref[0:1, :]
  s_ref[...] = acc[:, 0:256]
  bt_ref[...] = acc[:, 256:384]
  c_ref[...] = acc[:, 384:512]


def _mm3(h, wt, b8, br):
  n = h.shape[0]
  return pl.pallas_call(
      _mm3_body,
      grid=(n // br,),
      in_specs=[
          pl.BlockSpec((br, 128), lambda i: (i, 0)),
          pl.BlockSpec((128, 512), lambda i: (0, 0)),
          pl.BlockSpec((8, 512), lambda i: (0, 0)),
      ],
      out_specs=[
          pl.BlockSpec((br, 256), lambda i: (i, 0)),
          pl.BlockSpec((br, 128), lambda i: (i, 0)),
          pl.BlockSpec((br, 128), lambda i: (i, 0)),
      ],
      out_shape=[
          jax.ShapeDtypeStruct((n, 256), jnp.float32),
          jax.ShapeDtypeStruct((n, 128), jnp.float32),
          jax.ShapeDtypeStruct((n, 128), jnp.float32),
      ],
  )(h, wt, b8)


def _mm1_body(e_ref, w_ref, b_ref, o_ref):
  o_ref[...] = (
      jnp.dot(e_ref[...], w_ref[...], preferred_element_type=jnp.float32)
      + b_ref[0:1, :])


def _mm1_padded(ef, wt, b8, br, ep):
  n = ef.shape[0]
  last = n // br - 1
  return pl.pallas_call(
      _mm1_body,
      grid=(ep // br,),
      in_specs=[
          pl.BlockSpec((br, 128), lambda i: (jnp.minimum(i, last), 0)),
          pl.BlockSpec((128, 128), lambda i: (0, 0)),
          pl.BlockSpec((8, 128), lambda i: (0, 0)),
      ],
      out_specs=pl.BlockSpec((br, 128), lambda i: (i, 0)),
      out_shape=jax.ShapeDtypeStruct((ep, 128), jnp.float32),
  )(ef, wt, b8)


def _ln_silu(t, g_ref, b_ref):
  mu = jnp.mean(t, axis=-1, keepdims=True)
  var = jnp.mean((t - mu) * (t - mu), axis=-1, keepdims=True)
  tn = (t - mu) / jnp.sqrt(var + 1e-5) * g_ref[0:1, :] + b_ref[0:1, :]
  return tn * (1.0 / (1.0 + jnp.exp(-tn)))


def _post_node_body(x_ref, c_ref, ss_ref, g_ref, b_ref, o_ref):
  s0 = ss_ref[:, 0:128]
  s1 = ss_ref[:, 128:256]
  t = c_ref[...] + s1 / (s0 + 1e-6)
  o_ref[...] = x_ref[...] + _ln_silu(t, g_ref, b_ref)


def _post_node(x, c, ss, g8, b8, br):
  n = x.shape[0]
  return pl.pallas_call(
      _post_node_body,
      grid=(n // br,),
      in_specs=[
          pl.BlockSpec((br, 128), lambda i: (i, 0)),
          pl.BlockSpec((br, 128), lambda i: (i, 0)),
          pl.BlockSpec((br, 256), lambda i: (i, 0)),
          pl.BlockSpec((8, 128), lambda i: (0, 0)),
          pl.BlockSpec((8, 128), lambda i: (0, 0)),
      ],
      out_specs=pl.BlockSpec((br, 128), lambda i: (i, 0)),
      out_shape=jax.ShapeDtypeStruct((n, 128), jnp.float32),
  )(x, c, ss, g8, b8)


def _post_edge_body(e_ref, m_ref, g_ref, b_ref, o_ref):
  o_ref[...] = e_ref[...] + _ln_silu(m_ref[...], g_ref, b_ref)


def _post_edge(ef, m, g8, b8, br):
  n = ef.shape[0]
  return pl.pallas_call(
      _post_edge_body,
      grid=(n // br,),
      in_specs=[
          pl.BlockSpec((br, 128), lambda i: (i, 0)),
          pl.BlockSpec((br, 128), lambda i: (i, 0)),
          pl.BlockSpec((8, 128), lambda i: (0, 0)),
          pl.BlockSpec((8, 128), lambda i: (0, 0)),
      ],
      out_specs=pl.BlockSpec((br, 128), lambda i: (i, 0)),
      out_shape=jax.ShapeDtypeStruct((n, 128), jnp.float32),
  )(ef, m, g8, b8)


_SC_PARAMS = pltpu.CompilerParams(
    needs_layout_passes=False, use_tc_tiling_on_sc=False)


def _make_gate(ep):
  CH = 64
  per_w = ep // NW
  nch = per_w // CH
  mesh = plsc.VectorSubcoreMesh(core_axis_name="c", subcore_axis_name="s")

  buf_set = [
      pltpu.VMEM((CH,), jnp.int32),
      pltpu.VMEM((CH,), jnp.int32),
      pltpu.VMEM((CH, 256), jnp.float32),
      pltpu.VMEM((CH, 128), jnp.float32),
      pltpu.VMEM((CH, 128), jnp.float32),
      pltpu.VMEM((CH, 128), jnp.float32),
      pltpu.VMEM((CH * 16, 16), jnp.float32),
      pltpu.SemaphoreType.DMA,
      pltpu.SemaphoreType.DMA,
  ]

  @functools.partial(
      pl.kernel,
      out_type=[
          jax.ShapeDtypeStruct((ep, 128), jnp.float32),
          jax.ShapeDtypeStruct((ep * 16, 16), jnp.float32),
      ],
      mesh=mesh,
      compiler_params=_SC_PARAMS,
      scratch_types=buf_set + buf_set,
  )
  def gate(s_hbm, bt_hbm, el_hbm, src_hbm, dst_hbm, m_hbm, p_hbm,
           *bufs):
    wid = lax.axis_index("s") * NC + lax.axis_index("c")
    base = wid * per_w
    b0 = bufs[:9]
    b1 = bufs[9:]

    def start_in(ch, b):
      (sidx, didx, srows, brows, erows, m1b, pb, in_sem, out_sem) = b
      off = base + ch * CH
      pltpu.sync_copy(src_hbm.at[pl.ds(off, CH)], sidx)
      pltpu.sync_copy(dst_hbm.at[pl.ds(off, CH)], didx)
      pltpu.async_copy(s_hbm.at[sidx], srows, in_sem)
      pltpu.async_copy(bt_hbm.at[didx], brows, in_sem)
      pltpu.async_copy(el_hbm.at[pl.ds(off, CH)], erows, in_sem)

    def compute_out(ch, b):
      (sidx, didx, srows, brows, erows, m1b, pb, in_sem, out_sem) = b
      off = base + ch * CH
      @pl.when(ch >= 2)
      def _():
        pltpu.make_async_copy(m1b, m_hbm.at[pl.ds(off, CH)], out_sem).wait()
        pltpu.make_async_copy(pb, p_hbm.at[pl.ds(off * 16, CH * 16)],
                              out_sem).wait()

      pltpu.make_async_copy(s_hbm.at[sidx], srows, in_sem).wait()
      pltpu.make_async_copy(bt_hbm.at[didx], brows, in_sem).wait()
      pltpu.make_async_copy(el_hbm.at[pl.ds(off, CH)], erows, in_sem).wait()

      def row(j, c2):
        for k in range(8):
          sl = pl.ds(k * 16, 16)
          sh = pl.ds(128 + k * 16, 16)
          a = srows[j, sl]
          bh = srows[j, sh]
          m = a + brows[j, sl] + erows[j, sl]
          m1b[j, sl] = m
          sg = 1.0 / (1.0 + jnp.exp(-m))
          pb[j * 16 + k, :] = sg
          pb[j * 16 + 8 + k, :] = sg * bh
        return c2

      lax.fori_loop(0, CH, row, 0)
      pltpu.async_copy(m1b, m_hbm.at[pl.ds(off, CH)], out_sem)
      pltpu.async_copy(pb, p_hbm.at[pl.ds(off * 16, CH * 16)], out_sem)

    start_in(0, b0)

    def pair(t, carry):
      ch0 = t * 2
      start_in(ch0 + 1, b1)
      compute_out(ch0, b0)

      @pl.when(ch0 + 2 < nch)
      def _():
        start_in(ch0 + 2, b0)

      compute_out(ch0 + 1, b1)
      return carry

    lax.fori_loop(0, nch // 2, pair, 0)
    (_, _, _, _, _, m1b0, pb0, _, out_sem0) = b0
    (_, _, _, _, _, m1b1, pb1, _, out_sem1) = b1
    off_l0 = base + (nch - 2) * CH
    off_l1 = base + (nch - 1) * CH
    pltpu.make_async_copy(m1b0, m_hbm.at[pl.ds(off_l0, CH)], out_sem0).wait()
    pltpu.make_async_copy(pb0, p_hbm.at[pl.ds(off_l0 * 16, CH * 16)],
                          out_sem0).wait()
    pltpu.make_async_copy(m1b1, m_hbm.at[pl.ds(off_l1, CH)], out_sem1).wait()
    pltpu.make_async_copy(pb1, p_hbm.at[pl.ds(off_l1 * 16, CH * 16)],
                          out_sem1).wait()

  return gate


def _make_scatter(ep, nout, half, cn):
  share = ep // NS
  npass = half // cn
  WIN = 2048
  nwin = share // WIN
  BLK = 512
  REG = share + 544
  NWB = cn // 128
  mesh = plsc.VectorSubcoreMesh(core_axis_name="c", subcore_axis_name="s")

  @functools.partial(
      pl.kernel,
      out_type=jax.ShapeDtypeStruct((nout * 16, 16), jnp.float32),
      mesh=mesh,
      compiler_params=_SC_PARAMS,
      scratch_types=[
          pltpu.VMEM((WIN,), jnp.int32),
          pltpu.VMEM((WIN + 16,), jnp.int32),
          pltpu.VMEM((16,), jnp.int32),
          pltpu.VMEM((16, 16), jnp.int32),
          pltpu.VMEM((BLK + 16,), jnp.int32),
          pltpu.VMEM((4, 128), jnp.int32),
          pltpu.VMEM((BLK, 16), jnp.float32),
          pltpu.SemaphoreType.DMA,
          pltpu.VMEM((BLK + 16,), jnp.int32),
          pltpu.VMEM((4, 128), jnp.int32),
          pltpu.VMEM((BLK, 16), jnp.float32),
          pltpu.SemaphoreType.DMA,
          pltpu.VMEM((cn + 16, 16), jnp.float32),
          pltpu.VMEM((NWB, 128), jnp.int32),
          pltpu.SemaphoreType.DMA,
          pltpu.VMEM_SHARED((16, REG), jnp.int32),
          pltpu.VMEM_SHARED((16, 16), jnp.int32),
      ],
  )
  def scat(dst_hbm, p_hbm, out_hbm,
           dwin, wpk, cbuf, cvm,
           pkbA, ixbA, gbA, semgA,
           pkbB, ixbB, gbB, semgB,
           acc2, wbix, semw, pk_sp, cnt_sp):
    bufA = (pkbA, ixbA, gbA, semgA)
    bufB = (pkbB, ixbB, gbB, semgB)
    c = lax.axis_index("c")
    s = lax.axis_index("s")
    iota = lax.iota(jnp.int32, 16)
    zero16 = jnp.zeros((16,), jnp.float32)
    fill16 = jnp.full((16,), cn, jnp.int32)

    def one_pass(p, pcarry):
      base = pl.multiple_of(c * half + p * cn, 8)

      def az(j, carry):
        for q in range(8):
          acc2[j * 8 + q, :] = zero16
        return carry

      lax.fori_loop(0, (cn + 16) // 8, az, 0)

      def wloop(w, cntv):
        pltpu.sync_copy(dst_hbm.at[pl.ds(s * share + w * WIN, WIN)], dwin)

        def vec(v, wcv):
          d = dwin[pl.ds(v * 16, 16)]
          rel = d - base
          mask = (rel >= 0) & (rel < cn)
          pos = wcv - 1 + plsc.cumsum(mask.astype(jnp.int32))
          gid = s * share + w * WIN + v * 16 + iota
          pk = lax.shift_left(gid, 13) | rel
          plsc.store_scatter(wpk, [pos], pk, mask=mask)
          return wcv + plsc.all_reduce_population_count(mask)

        wcv = lax.fori_loop(0, WIN // 16, vec, jnp.zeros((16,), jnp.int32))
        wc = wcv[0]
        wpk[pl.ds(wc, 16)] = fill16
        cnt = pl.multiple_of(cntv[0], 8)
        pltpu.sync_copy(wpk, pk_sp.at[s, pl.ds(cnt, WIN + 16)])
        return cntv + ((wcv + 7) & (-8))

      cntv = lax.fori_loop(0, nwin, wloop, jnp.zeros((16,), jnp.int32))
      cnt = pl.multiple_of(cntv[0], 8)
      for t in range(BLK // 16):
        wpk[pl.ds(t * 16, 16)] = fill16
      pltpu.sync_copy(wpk.at[pl.ds(0, BLK)], pk_sp.at[s, pl.ds(cnt, BLK)])
      cbuf[pl.ds(0, 16)] = cntv
      pltpu.sync_copy(cbuf, cnt_sp.at[s])
      plsc.subcore_barrier()

      pltpu.sync_copy(cnt_sp, cvm)

      def fetch(t, g, buf):
        pkb, ixb, gb, semg = buf
        pltpu.sync_copy(pk_sp.at[t, pl.ds(g * BLK, BLK)],
                        pkb.at[pl.ds(0, BLK)])
        for q in range(4):
          for q2 in range(8):
            pk = pkb[pl.ds(q * 128 + q2 * 16, 16)]
            eid = lax.shift_right_logical(pk, 13)
            ixb[q, pl.ds(q2 * 16, 16)] = eid * 16 + s
        for q in range(4):
          pltpu.async_copy(p_hbm.at[ixb.at[q]],
                           gb.at[pl.ds(q * 128, 128)], semg)

      def drain_acc(buf):
        pkb, ixb, gb, semg = buf
        for q in range(4):
          pltpu.make_async_copy(p_hbm.at[ixb.at[q]],
                                gb.at[pl.ds(q * 128, 128)], semg).wait()

        def step(i, c2):
          rv = pkb[pl.ds(i, 16)] & 8191
          plsc.addupdate_scatter(acc2, [(iota & 0) + rv[0], iota], gb[i, :])
          return c2

        lax.fori_loop(0, BLK, step, 0)

      def walk_t(t, tcarry):
        cnt_t = cvm[t, :][0]
        nb = lax.div(cnt_t + BLK - 1, BLK)

        @pl.when(nb > 0)
        def _():
          fetch(t, 0, bufA)

        def pairq(q, carry):
          g0 = 2 * q

          @pl.when(g0 + 1 < nb)
          def _():
            fetch(t, g0 + 1, bufB)

          drain_acc(bufA)

          @pl.when(g0 + 2 < nb)
          def _():
            fetch(t, g0 + 2, bufA)

          @pl.when(g0 + 1 < nb)
          def _():
            drain_acc(bufB)

          return carry

        lax.fori_loop(0, lax.div(nb + 1, 2), pairq, 0)
        return tcarry

      lax.fori_loop(0, NS, walk_t, 0)

      ob16 = (base + s * 0) * 16
      for sb in range(NWB):
        for q2 in range(8):
          rr = sb * 128 + q2 * 16 + iota
          wbix[sb, pl.ds(q2 * 16, 16)] = (base + rr) * 16 + s
        pltpu.async_copy(acc2.at[pl.ds(sb * 128, 128)],
                         out_hbm.at[wbix.at[sb]], semw)
      for sb in range(NWB):
        pltpu.make_async_copy(acc2.at[pl.ds(sb * 128, 128)],
                              out_hbm.at[wbix.at[sb]], semw).wait()
      plsc.subcore_barrier()
      return pcarry

    lax.fori_loop(0, npass, one_pass, 0)

  return scat


def _egc_layer(h, ef, src, dst, p, ep, nout, half, cn, br_n, br_e):
  n = h.shape[0]
  e = ef.shape[0]

  wt = jnp.concatenate(
      [p['W_src_gate'].T, p['W_dst_update'].T,
       p['W_dst_gate'].T, p['W_src_update'].T], axis=1)
  bcat = jnp.concatenate(
      [p['b_src_gate'], p['b_dst_update'],
       p['b_dst_gate'], p['b_src_update']])
  b8 = jnp.tile(bcat[None, :], (8, 1))
  s_tab, bt_tab, c_tab = _mm3(h, wt, b8, br_n)

  wet = p['W_edge_gate'].T
  be8 = jnp.tile(p['b_edge_gate'][None, :], (8, 1))
  elin = _mm1_padded(ef, wet, be8, br_e, ep)

  npad = ep - e
  fill = jnp.arange(npad, dtype=jnp.int32) % n
  src_g = jnp.concatenate([src, fill])
  dst_g = jnp.concatenate([dst, fill])
  dst_s = jnp.concatenate([dst, jnp.full((npad,), 1 << 20, jnp.int32)])

  m1, pay = _make_gate(ep)(s_tab, bt_tab, elin, src_g, dst_g)
  ss16 = _make_scatter(ep, nout, half, cn)(dst_s, pay)
  ss = jnp.reshape(ss16, (nout, 256))

  g_n8 = jnp.tile(p['g_nodes'][None, :], (8, 1))
  b_n8 = jnp.tile(p['b_nodes'][None, :], (8, 1))
  g_e8 = jnp.tile(p['g_edges'][None, :], (8, 1))
  b_e8 = jnp.tile(p['b_edges'][None, :], (8, 1))

  x_out = _post_node(h, c_tab, ss, g_n8, b_n8, br_n)
  y_out = _post_edge(ef, m1, g_e8, b_e8, br_e)
  return x_out, y_out


def kernel(x, y, z, nu_params, eu_params, edge_index, lg_edge_index):
  src, dst = edge_index[0], edge_index[1]
  lsrc, ldst = lg_edge_index[0], lg_edge_index[1]

  x_out, m = _egc_layer(
      x, y, src, dst, nu_params,
      ep=163840, nout=10240, half=5120, cn=2560,
      br_n=400, br_e=640)

  y_out, z_out = _egc_layer(
      m, z, lsrc, ldst, eu_params,
      ep=327680, nout=163840, half=81920, cn=4096,
      br_n=640, br_e=640)

  return (x_out, y_out, z_out)


# speedup vs baseline: 1.9664x; 1.9664x over previous
import functools

import jax
import jax.numpy as jnp
from jax import lax
from jax.experimental import pallas as pl
from jax.experimental.pallas import tpu as pltpu
from jax.experimental.pallas import tpu_sc as plsc

NC = 2
NS = 16
NW = NC * NS


def _mm3_body(h_ref, w_ref, b_ref, s_ref, bt_ref, c_ref):
  acc = jnp.dot(h_ref[...], w_ref[...], preferred_element_type=jnp.float32)
  acc = acc + b_ref[0:1, :]
  s_ref[...] = acc[:, 0:256]
  bt_ref[...] = acc[:, 256:384]
  c_ref[...] = acc[:, 384:512]


def _mm3(h, wt, b8, br):
  n = h.shape[0]
  return pl.pallas_call(
      _mm3_body,
      grid=(n // br,),
      in_specs=[
          pl.BlockSpec((br, 128), lambda i: (i, 0)),
          pl.BlockSpec((128, 512), lambda i: (0, 0)),
          pl.BlockSpec((8, 512), lambda i: (0, 0)),
      ],
      out_specs=[
          pl.BlockSpec((br, 256), lambda i: (i, 0)),
          pl.BlockSpec((br, 128), lambda i: (i, 0)),
          pl.BlockSpec((br, 128), lambda i: (i, 0)),
      ],
      out_shape=[
          jax.ShapeDtypeStruct((n, 256), jnp.float32),
          jax.ShapeDtypeStruct((n, 128), jnp.float32),
          jax.ShapeDtypeStruct((n, 128), jnp.float32),
      ],
  )(h, wt, b8)


def _mm1_body(e_ref, w_ref, b_ref, o_ref):
  o_ref[...] = (
      jnp.dot(e_ref[...], w_ref[...], preferred_element_type=jnp.float32)
      + b_ref[0:1, :])


def _mm1_padded(ef, wt, b8, br, ep):
  n = ef.shape[0]
  last = n // br - 1
  return pl.pallas_call(
      _mm1_body,
      grid=(ep // br,),
      in_specs=[
          pl.BlockSpec((br, 128), lambda i: (jnp.minimum(i, last), 0)),
          pl.BlockSpec((128, 128), lambda i: (0, 0)),
          pl.BlockSpec((8, 128), lambda i: (0, 0)),
      ],
      out_specs=pl.BlockSpec((br, 128), lambda i: (i, 0)),
      out_shape=jax.ShapeDtypeStruct((ep, 128), jnp.float32),
  )(ef, wt, b8)


def _ln_silu(t, g_ref, b_ref):
  mu = jnp.mean(t, axis=-1, keepdims=True)
  var = jnp.mean((t - mu) * (t - mu), axis=-1, keepdims=True)
  tn = (t - mu) / jnp.sqrt(var + 1e-5) * g_ref[0:1, :] + b_ref[0:1, :]
  return tn * (1.0 / (1.0 + jnp.exp(-tn)))


def _post_node_body(x_ref, c_ref, ss_ref, g_ref, b_ref, o_ref):
  s0 = ss_ref[:, 0:128]
  s1 = ss_ref[:, 128:256]
  t = c_ref[...] + s1 / (s0 + 1e-6)
  o_ref[...] = x_ref[...] + _ln_silu(t, g_ref, b_ref)


def _post_node(x, c, ss, g8, b8, br):
  n = x.shape[0]
  return pl.pallas_call(
      _post_node_body,
      grid=(n // br,),
      in_specs=[
          pl.BlockSpec((br, 128), lambda i: (i, 0)),
          pl.BlockSpec((br, 128), lambda i: (i, 0)),
          pl.BlockSpec((br, 256), lambda i: (i, 0)),
          pl.BlockSpec((8, 128), lambda i: (0, 0)),
          pl.BlockSpec((8, 128), lambda i: (0, 0)),
      ],
      out_specs=pl.BlockSpec((br, 128), lambda i: (i, 0)),
      out_shape=jax.ShapeDtypeStruct((n, 128), jnp.float32),
  )(x, c, ss, g8, b8)


def _post_edge_body(e_ref, m_ref, g_ref, b_ref, o_ref):
  o_ref[...] = e_ref[...] + _ln_silu(m_ref[...], g_ref, b_ref)


def _post_edge(ef, m, g8, b8, br):
  n = ef.shape[0]
  return pl.pallas_call(
      _post_edge_body,
      grid=(n // br,),
      in_specs=[
          pl.BlockSpec((br, 128), lambda i: (i, 0)),
          pl.BlockSpec((br, 128), lambda i: (i, 0)),
          pl.BlockSpec((8, 128), lambda i: (0, 0)),
          pl.BlockSpec((8, 128), lambda i: (0, 0)),
      ],
      out_specs=pl.BlockSpec((br, 128), lambda i: (i, 0)),
      out_shape=jax.ShapeDtypeStruct((n, 128), jnp.float32),
  )(ef, m, g8, b8)


_SC_PARAMS = pltpu.CompilerParams(
    needs_layout_passes=False, use_tc_tiling_on_sc=False)


def _make_gate(ep):
  CH = 64
  per_w = ep // NW
  nch = per_w // CH
  mesh = plsc.VectorSubcoreMesh(core_axis_name="c", subcore_axis_name="s")

  buf_set = [
      pltpu.VMEM((CH,), jnp.int32),
      pltpu.VMEM((CH,), jnp.int32),
      pltpu.VMEM((CH, 256), jnp.float32),
      pltpu.VMEM((CH, 128), jnp.float32),
      pltpu.VMEM((CH, 128), jnp.float32),
      pltpu.VMEM((CH, 128), jnp.float32),
      pltpu.VMEM((CH * 16, 16), jnp.float32),
      pltpu.SemaphoreType.DMA,
      pltpu.SemaphoreType.DMA,
  ]

  @functools.partial(
      pl.kernel,
      out_type=[
          jax.ShapeDtypeStruct((ep, 128), jnp.float32),
          jax.ShapeDtypeStruct((ep * 16, 16), jnp.float32),
      ],
      mesh=mesh,
      compiler_params=_SC_PARAMS,
      scratch_types=buf_set + buf_set,
  )
  def gate(s_hbm, bt_hbm, el_hbm, src_hbm, dst_hbm, m_hbm, p_hbm,
           *bufs):
    wid = lax.axis_index("s") * NC + lax.axis_index("c")
    base = wid * per_w
    b0 = bufs[:9]
    b1 = bufs[9:]

    def start_in(ch, b):
      (sidx, didx, srows, brows, erows, m1b, pb, in_sem, out_sem) = b
      off = base + ch * CH
      pltpu.sync_copy(src_hbm.at[pl.ds(off, CH)], sidx)
      pltpu.sync_copy(dst_hbm.at[pl.ds(off, CH)], didx)
      pltpu.async_copy(s_hbm.at[sidx], srows, in_sem)
      pltpu.async_copy(bt_hbm.at[didx], brows, in_sem)
      pltpu.async_copy(el_hbm.at[pl.ds(off, CH)], erows, in_sem)

    def compute_out(ch, b):
      (sidx, didx, srows, brows, erows, m1b, pb, in_sem, out_sem) = b
      off = base + ch * CH
      @pl.when(ch >= 2)
      def _():
        pltpu.make_async_copy(m1b, m_hbm.at[pl.ds(off, CH)], out_sem).wait()
        pltpu.make_async_copy(pb, p_hbm.at[pl.ds(off * 16, CH * 16)],
                              out_sem).wait()

      pltpu.make_async_copy(s_hbm.at[sidx], srows, in_sem).wait()
      pltpu.make_async_copy(bt_hbm.at[didx], brows, in_sem).wait()
      pltpu.make_async_copy(el_hbm.at[pl.ds(off, CH)], erows, in_sem).wait()

      def row(j, c2):
        for k in range(8):
          sl = pl.ds(k * 16, 16)
          sh = pl.ds(128 + k * 16, 16)
          a = srows[j, sl]
          bh = srows[j, sh]
          m = a + brows[j, sl] + erows[j, sl]
          m1b[j, sl] = m
          sg = 1.0 / (1.0 + jnp.exp(-m))
          pb[j * 16 + k, :] = sg
          pb[j * 16 + 8 + k, :] = sg * bh
        return c2

      lax.fori_loop(0, CH, row, 0)
      pltpu.async_copy(m1b, m_hbm.at[pl.ds(off, CH)], out_sem)
      pltpu.async_copy(pb, p_hbm.at[pl.ds(off * 16, CH * 16)], out_sem)

    start_in(0, b0)

    def pair(t, carry):
      ch0 = t * 2
      start_in(ch0 + 1, b1)
      compute_out(ch0, b0)

      @pl.when(ch0 + 2 < nch)
      def _():
        start_in(ch0 + 2, b0)

      compute_out(ch0 + 1, b1)
      return carry

    lax.fori_loop(0, nch // 2, pair, 0)
    (_, _, _, _, _, m1b0, pb0, _, out_sem0) = b0
    (_, _, _, _, _, m1b1, pb1, _, out_sem1) = b1
    off_l0 = base + (nch - 2) * CH
    off_l1 = base + (nch - 1) * CH
    pltpu.make_async_copy(m1b0, m_hbm.at[pl.ds(off_l0, CH)], out_sem0).wait()
    pltpu.make_async_copy(pb0, p_hbm.at[pl.ds(off_l0 * 16, CH * 16)],
                          out_sem0).wait()
    pltpu.make_async_copy(m1b1, m_hbm.at[pl.ds(off_l1, CH)], out_sem1).wait()
    pltpu.make_async_copy(pb1, p_hbm.at[pl.ds(off_l1 * 16, CH * 16)],
                          out_sem1).wait()

  return gate


def _make_scatter(ep, nout, half, cn):
  share = ep // NS
  npass = half // cn
  WIN = 1024
  nwin = share // WIN
  REG = share + 144
  mesh = plsc.VectorSubcoreMesh(core_axis_name="c", subcore_axis_name="s")

  @functools.partial(
      pl.kernel,
      out_type=jax.ShapeDtypeStruct((nout, 256), jnp.float32),
      mesh=mesh,
      compiler_params=_SC_PARAMS,
      scratch_types=[
          pltpu.VMEM((WIN,), jnp.int32),
          pltpu.VMEM((WIN + 16,), jnp.int32),
          pltpu.VMEM((WIN + 16,), jnp.int32),
          pltpu.VMEM((16,), jnp.int32),
          pltpu.VMEM((16, 16), jnp.int32),
          pltpu.VMEM((144,), jnp.int32),
          pltpu.VMEM((144,), jnp.int32),
          pltpu.VMEM((128,), jnp.int32),
          pltpu.VMEM((128, 16), jnp.float32),
          pltpu.SemaphoreType.DMA,
          pltpu.SemaphoreType.DMA,
          pltpu.VMEM((144,), jnp.int32),
          pltpu.VMEM((144,), jnp.int32),
          pltpu.VMEM((128,), jnp.int32),
          pltpu.VMEM((128, 16), jnp.float32),
          pltpu.SemaphoreType.DMA,
          pltpu.SemaphoreType.DMA,
          pltpu.VMEM((cn + 16, 16), jnp.float32),
          pltpu.VMEM((16, 256), jnp.float32),
          pltpu.VMEM_SHARED((256, 256), jnp.float32),
          pltpu.VMEM_SHARED((16, REG), jnp.int32),
          pltpu.VMEM_SHARED((16, REG), jnp.int32),
          pltpu.VMEM_SHARED((16, 16), jnp.int32),
          pltpu.SemaphoreType.DMA,
      ],
  )
  def scat(dst_hbm, p_hbm, out_hbm,
           dwin, wids, wlds, cbuf, cvm,
           idbA, ldbA, ixbA, gbA, semgA, semlA,
           idbB, ldbB, ixbB, gbB, semgB, semlB,
           acc2, obuf, stg, ids_sp, lds_sp, cnt_sp, sem):
    bufA = (idbA, ldbA, ixbA, gbA, semgA, semlA)
    bufB = (idbB, ldbB, ixbB, gbB, semgB, semlB)
    c = lax.axis_index("c")
    s = lax.axis_index("s")
    iota = lax.iota(jnp.int32, 16)
    zero16 = jnp.zeros((16,), jnp.float32)
    zero16i = jnp.zeros((16,), jnp.int32)
    trash16 = jnp.full((16,), cn, jnp.int32)

    def one_pass(p, pcarry):
      base = pl.multiple_of(c * half + p * cn, 8)

      def az(j, carry):
        for q in range(8):
          acc2[j * 8 + q, :] = zero16
        return carry

      lax.fori_loop(0, (cn + 16) // 8, az, 0)

      def wloop(w, cntv):
        pltpu.sync_copy(dst_hbm.at[pl.ds(s * share + w * WIN, WIN)], dwin)

        def vec(v, wcv):
          d = dwin[pl.ds(v * 16, 16)]
          rel = d - base
          mask = (rel >= 0) & (rel < cn)
          pos = wcv - 1 + plsc.cumsum(mask.astype(jnp.int32))
          gid = s * share + w * WIN + v * 16 + iota
          plsc.store_scatter(wids, [pos], gid, mask=mask)
          plsc.store_scatter(wlds, [pos], rel, mask=mask)
          return wcv + plsc.all_reduce_population_count(mask)

        wcv = lax.fori_loop(0, WIN // 16, vec, jnp.zeros((16,), jnp.int32))
        wc = wcv[0]
        wids[pl.ds(wc, 16)] = zero16i
        wlds[pl.ds(wc, 16)] = trash16
        cnt = pl.multiple_of(cntv[0], 8)
        pltpu.sync_copy(wids, ids_sp.at[s, pl.ds(cnt, WIN + 16)])
        pltpu.sync_copy(wlds, lds_sp.at[s, pl.ds(cnt, WIN + 16)])
        return cntv + ((wcv + 7) & (-8))

      cntv = lax.fori_loop(0, nwin, wloop, jnp.zeros((16,), jnp.int32))
      cnt = pl.multiple_of(cntv[0], 8)
      for t in range(8):
        wids[pl.ds(t * 16, 16)] = zero16i
        wlds[pl.ds(t * 16, 16)] = trash16
      pltpu.sync_copy(wids.at[pl.ds(0, 128)], ids_sp.at[s, pl.ds(cnt, 128)])
      pltpu.sync_copy(wlds.at[pl.ds(0, 128)], lds_sp.at[s, pl.ds(cnt, 128)])
      cbuf[pl.ds(0, 16)] = cntv
      pltpu.sync_copy(cbuf, cnt_sp.at[s])
      plsc.subcore_barrier()

      pltpu.sync_copy(cnt_sp, cvm)

      def fetch(t, g, buf):
        idb, ldb, ixb, gb, semg, seml = buf
        pltpu.sync_copy(ids_sp.at[t, pl.ds(g * 128, 128)],
                        idb.at[pl.ds(0, 128)])
        pltpu.async_copy(lds_sp.at[t, pl.ds(g * 128, 128)],
                         ldb.at[pl.ds(0, 128)], seml)
        for q in range(8):
          ixb[pl.ds(q * 16, 16)] = idb[pl.ds(q * 16, 16)] * 16 + s
        pltpu.async_copy(p_hbm.at[ixb], gb, semg)

      def drain_acc(buf):
        idb, ldb, ixb, gb, semg, seml = buf
        pltpu.make_async_copy(lds_sp.at[0, pl.ds(0, 128)],
                              ldb.at[pl.ds(0, 128)], seml).wait()
        pltpu.make_async_copy(p_hbm.at[ixb], gb, semg).wait()

        def step(i, c2):
          rv = ldb[pl.ds(i, 16)]
          plsc.addupdate_scatter(acc2, [zero16i + rv[0], iota], gb[i, :])
          return c2

        lax.fori_loop(0, 128, step, 0)

      def walk_t(t, tcarry):
        cnt_t = cvm[t, :][0]
        nb = lax.div(cnt_t + 127, 128)

        @pl.when(nb > 0)
        def _():
          fetch(t, 0, bufA)

        def pairq(q, carry):
          g0 = 2 * q

          @pl.when(g0 + 1 < nb)
          def _():
            fetch(t, g0 + 1, bufB)

          drain_acc(bufA)

          @pl.when(g0 + 2 < nb)
          def _():
            fetch(t, g0 + 2, bufA)

          @pl.when(g0 + 1 < nb)
          def _():
            drain_acc(bufB)

          return carry

        lax.fori_loop(0, lax.div(nb + 1, 2), pairq, 0)
        return tcarry

      lax.fori_loop(0, NS, walk_t, 0)
      plsc.subcore_barrier()

      SB = 256
      rpt = SB // NS

      def unint(sb, carry):
        pltpu.sync_copy(acc2.at[pl.ds(pl.multiple_of(sb * SB, 8), SB)],
                        stg.at[:, pl.ds(s * 16, 16)])
        plsc.subcore_barrier()
        r0 = s * rpt
        pltpu.sync_copy(stg.at[pl.ds(r0, rpt)], obuf)
        ofs = pl.multiple_of(base + sb * SB + r0, 8)
        pltpu.sync_copy(obuf, out_hbm.at[pl.ds(ofs, rpt)])
        plsc.subcore_barrier()
        return carry

      lax.fori_loop(0, cn // SB, unint, 0)
      return pcarry

    lax.fori_loop(0, npass, one_pass, 0)

  return scat


def _egc_layer(h, ef, src, dst, p, ep, nout, half, cn, br_n, br_e):
  n = h.shape[0]
  e = ef.shape[0]

  wt = jnp.concatenate(
      [p['W_src_gate'].T, p['W_dst_update'].T,
       p['W_dst_gate'].T, p['W_src_update'].T], axis=1)
  bcat = jnp.concatenate(
      [p['b_src_gate'], p['b_dst_update'],
       p['b_dst_gate'], p['b_src_update']])
  b8 = jnp.tile(bcat[None, :], (8, 1))
  s_tab, bt_tab, c_tab = _mm3(h, wt, b8, br_n)

  wet = p['W_edge_gate'].T
  be8 = jnp.tile(p['b_edge_gate'][None, :], (8, 1))
  elin = _mm1_padded(ef, wet, be8, br_e, ep)

  npad = ep - e
  fill = jnp.arange(npad, dtype=jnp.int32) % n
  src_g = jnp.concatenate([src, fill])
  dst_g = jnp.concatenate([dst, fill])
  dst_s = jnp.concatenate([dst, jnp.full((npad,), 1 << 20, jnp.int32)])

  m1, pay = _make_gate(ep)(s_tab, bt_tab, elin, src_g, dst_g)
  ss = _make_scatter(ep, nout, half, cn)(dst_s, pay)

  g_n8 = jnp.tile(p['g_nodes'][None, :], (8, 1))
  b_n8 = jnp.tile(p['b_nodes'][None, :], (8, 1))
  g_e8 = jnp.tile(p['g_edges'][None, :], (8, 1))
  b_e8 = jnp.tile(p['b_edges'][None, :], (8, 1))

  x_out = _post_node(h, c_tab, ss, g_n8, b_n8, br_n)
  y_out = _post_edge(ef, m1, g_e8, b_e8, br_e)
  return x_out, y_out


def kernel(x, y, z, nu_params, eu_params, edge_index, lg_edge_index):
  src, dst = edge_index[0], edge_index[1]
  lsrc, ldst = lg_edge_index[0], lg_edge_index[1]

  x_out, m = _egc_layer(
      x, y, src, dst, nu_params,
      ep=163840, nout=10240, half=5120, cn=5120,
      br_n=400, br_e=640)

  y_out, z_out = _egc_layer(
      m, z, lsrc, ldst, eu_params,
      ep=327680, nout=163840, half=81920, cn=4096,
      br_n=640, br_e=640)

  return (x_out, y_out, z_out)


# speedup vs baseline: 2.0171x; 1.0257x over previous
import functools

import jax
import jax.numpy as jnp
from jax import lax
from jax.experimental import pallas as pl
from jax.experimental.pallas import tpu as pltpu
from jax.experimental.pallas import tpu_sc as plsc

NC = 2
NS = 16
NW = NC * NS


def _mm3_body(h_ref, w_ref, b_ref, s_ref, bt_ref, c_ref):
  acc = jnp.dot(h_ref[...], w_ref[...], preferred_element_type=jnp.float32)
  acc = acc + b_ref[0:1, :]
  s_ref[...] = acc[:, 0:256]
  bt_ref[...] = acc[:, 256:384]
  c_ref[...] = acc[:, 384:512]


def _mm3(h, wt, b8, br):
  n = h.shape[0]
  return pl.pallas_call(
      _mm3_body,
      grid=(n // br,),
      in_specs=[
          pl.BlockSpec((br, 128), lambda i: (i, 0)),
          pl.BlockSpec((128, 512), lambda i: (0, 0)),
          pl.BlockSpec((8, 512), lambda i: (0, 0)),
      ],
      out_specs=[
          pl.BlockSpec((br, 256), lambda i: (i, 0)),
          pl.BlockSpec((br, 128), lambda i: (i, 0)),
          pl.BlockSpec((br, 128), lambda i: (i, 0)),
      ],
      out_shape=[
          jax.ShapeDtypeStruct((n, 256), jnp.float32),
          jax.ShapeDtypeStruct((n, 128), jnp.float32),
          jax.ShapeDtypeStruct((n, 128), jnp.float32),
      ],
  )(h, wt, b8)


def _mm1_body(e_ref, w_ref, b_ref, o_ref):
  o_ref[...] = (
      jnp.dot(e_ref[...], w_ref[...], preferred_element_type=jnp.float32)
      + b_ref[0:1, :])


def _mm1_padded(ef, wt, b8, br, ep):
  n = ef.shape[0]
  last = n // br - 1
  return pl.pallas_call(
      _mm1_body,
      grid=(ep // br,),
      in_specs=[
          pl.BlockSpec((br, 128), lambda i: (jnp.minimum(i, last), 0)),
          pl.BlockSpec((128, 128), lambda i: (0, 0)),
          pl.BlockSpec((8, 128), lambda i: (0, 0)),
      ],
      out_specs=pl.BlockSpec((br, 128), lambda i: (i, 0)),
      out_shape=jax.ShapeDtypeStruct((ep, 128), jnp.float32),
  )(ef, wt, b8)


def _ln_silu(t, g_ref, b_ref):
  mu = jnp.mean(t, axis=-1, keepdims=True)
  var = jnp.mean((t - mu) * (t - mu), axis=-1, keepdims=True)
  tn = (t - mu) / jnp.sqrt(var + 1e-5) * g_ref[0:1, :] + b_ref[0:1, :]
  return tn * (1.0 / (1.0 + jnp.exp(-tn)))


def _post_node_body(x_ref, c_ref, ss_ref, g_ref, b_ref, o_ref):
  s0 = ss_ref[:, 0:128]
  s1 = ss_ref[:, 128:256]
  t = c_ref[...] + s1 / (s0 + 1e-6)
  o_ref[...] = x_ref[...] + _ln_silu(t, g_ref, b_ref)


def _post_node(x, c, ss, g8, b8, br):
  n = x.shape[0]
  return pl.pallas_call(
      _post_node_body,
      grid=(n // br,),
      in_specs=[
          pl.BlockSpec((br, 128), lambda i: (i, 0)),
          pl.BlockSpec((br, 128), lambda i: (i, 0)),
          pl.BlockSpec((br, 256), lambda i: (i, 0)),
          pl.BlockSpec((8, 128), lambda i: (0, 0)),
          pl.BlockSpec((8, 128), lambda i: (0, 0)),
      ],
      out_specs=pl.BlockSpec((br, 128), lambda i: (i, 0)),
      out_shape=jax.ShapeDtypeStruct((n, 128), jnp.float32),
  )(x, c, ss, g8, b8)


def _post_edge_body(e_ref, m_ref, g_ref, b_ref, o_ref):
  o_ref[...] = e_ref[...] + _ln_silu(m_ref[...], g_ref, b_ref)


def _post_edge(ef, m, g8, b8, br):
  n = ef.shape[0]
  return pl.pallas_call(
      _post_edge_body,
      grid=(n // br,),
      in_specs=[
          pl.BlockSpec((br, 128), lambda i: (i, 0)),
          pl.BlockSpec((br, 128), lambda i: (i, 0)),
          pl.BlockSpec((8, 128), lambda i: (0, 0)),
          pl.BlockSpec((8, 128), lambda i: (0, 0)),
      ],
      out_specs=pl.BlockSpec((br, 128), lambda i: (i, 0)),
      out_shape=jax.ShapeDtypeStruct((n, 128), jnp.float32),
  )(ef, m, g8, b8)


_SC_PARAMS = pltpu.CompilerParams(
    needs_layout_passes=False, use_tc_tiling_on_sc=False)


def _make_gate(ep):
  CH = 64
  per_w = ep // NW
  nch = per_w // CH
  mesh = plsc.VectorSubcoreMesh(core_axis_name="c", subcore_axis_name="s")

  buf_set = [
      pltpu.VMEM((CH,), jnp.int32),
      pltpu.VMEM((CH,), jnp.int32),
      pltpu.VMEM((CH, 256), jnp.float32),
      pltpu.VMEM((CH, 128), jnp.float32),
      pltpu.VMEM((CH, 128), jnp.float32),
      pltpu.VMEM((CH, 128), jnp.float32),
      pltpu.VMEM((CH * 16, 16), jnp.float32),
      pltpu.SemaphoreType.DMA,
      pltpu.SemaphoreType.DMA,
  ]

  @functools.partial(
      pl.kernel,
      out_type=[
          jax.ShapeDtypeStruct((ep, 128), jnp.float32),
          jax.ShapeDtypeStruct((ep * 16, 16), jnp.float32),
      ],
      mesh=mesh,
      compiler_params=_SC_PARAMS,
      scratch_types=buf_set + buf_set,
  )
  def gate(s_hbm, bt_hbm, el_hbm, src_hbm, dst_hbm, m_hbm, p_hbm,
           *bufs):
    wid = lax.axis_index("s") * NC + lax.axis_index("c")
    base = wid * per_w
    b0 = bufs[:9]
    b1 = bufs[9:]

    def start_in(ch, b):
      (sidx, didx, srows, brows, erows, m1b, pb, in_sem, out_sem) = b
      off = base + ch * CH
      pltpu.sync_copy(src_hbm.at[pl.ds(off, CH)], sidx)
      pltpu.sync_copy(dst_hbm.at[pl.ds(off, CH)], didx)
      pltpu.async_copy(s_hbm.at[sidx], srows, in_sem)
      pltpu.async_copy(bt_hbm.at[didx], brows, in_sem)
      pltpu.async_copy(el_hbm.at[pl.ds(off, CH)], erows, in_sem)

    def compute_out(ch, b):
      (sidx, didx, srows, brows, erows, m1b, pb, in_sem, out_sem) = b
      off = base + ch * CH
      @pl.when(ch >= 2)
      def _():
        pltpu.make_async_copy(m1b, m_hbm.at[pl.ds(off, CH)], out_sem).wait()
        pltpu.make_async_copy(pb, p_hbm.at[pl.ds(off * 16, CH * 16)],
                              out_sem).wait()

      pltpu.make_async_copy(s_hbm.at[sidx], srows, in_sem).wait()
      pltpu.make_async_copy(bt_hbm.at[didx], brows, in_sem).wait()
      pltpu.make_async_copy(el_hbm.at[pl.ds(off, CH)], erows, in_sem).wait()

      def row(j, c2):
        for k in range(8):
          sl = pl.ds(k * 16, 16)
          sh = pl.ds(128 + k * 16, 16)
          a = srows[j, sl]
          bh = srows[j, sh]
          m = a + brows[j, sl] + erows[j, sl]
          m1b[j, sl] = m
          sg = 1.0 / (1.0 + jnp.exp(-m))
          pb[j * 16 + k, :] = sg
          pb[j * 16 + 8 + k, :] = sg * bh
        return c2

      lax.fori_loop(0, CH, row, 0)
      pltpu.async_copy(m1b, m_hbm.at[pl.ds(off, CH)], out_sem)
      pltpu.async_copy(pb, p_hbm.at[pl.ds(off * 16, CH * 16)], out_sem)

    start_in(0, b0)

    def pair(t, carry):
      ch0 = t * 2
      start_in(ch0 + 1, b1)
      compute_out(ch0, b0)

      @pl.when(ch0 + 2 < nch)
      def _():
        start_in(ch0 + 2, b0)

      compute_out(ch0 + 1, b1)
      return carry

    lax.fori_loop(0, nch // 2, pair, 0)
    (_, _, _, _, _, m1b0, pb0, _, out_sem0) = b0
    (_, _, _, _, _, m1b1, pb1, _, out_sem1) = b1
    off_l0 = base + (nch - 2) * CH
    off_l1 = base + (nch - 1) * CH
    pltpu.make_async_copy(m1b0, m_hbm.at[pl.ds(off_l0, CH)], out_sem0).wait()
    pltpu.make_async_copy(pb0, p_hbm.at[pl.ds(off_l0 * 16, CH * 16)],
                          out_sem0).wait()
    pltpu.make_async_copy(m1b1, m_hbm.at[pl.ds(off_l1, CH)], out_sem1).wait()
    pltpu.make_async_copy(pb1, p_hbm.at[pl.ds(off_l1 * 16, CH * 16)],
                          out_sem1).wait()

  return gate


def _make_scatter(ep, nout, half, cn):
  share = ep // NS
  npass = half // cn
  WIN = 2048
  nwin = share // WIN
  REG = share + 144
  mesh = plsc.VectorSubcoreMesh(core_axis_name="c", subcore_axis_name="s")

  @functools.partial(
      pl.kernel,
      out_type=jax.ShapeDtypeStruct((nout, 256), jnp.float32),
      mesh=mesh,
      compiler_params=_SC_PARAMS,
      scratch_types=[
          pltpu.VMEM((WIN,), jnp.int32),
          pltpu.VMEM((WIN + 16,), jnp.int32),
          pltpu.VMEM((WIN + 16,), jnp.int32),
          pltpu.VMEM((16,), jnp.int32),
          pltpu.VMEM((16, 16), jnp.int32),
          pltpu.VMEM((144,), jnp.int32),
          pltpu.VMEM((144,), jnp.int32),
          pltpu.VMEM((128,), jnp.int32),
          pltpu.VMEM((128, 16), jnp.float32),
          pltpu.SemaphoreType.DMA,
          pltpu.SemaphoreType.DMA,
          pltpu.VMEM((144,), jnp.int32),
          pltpu.VMEM((144,), jnp.int32),
          pltpu.VMEM((128,), jnp.int32),
          pltpu.VMEM((128, 16), jnp.float32),
          pltpu.SemaphoreType.DMA,
          pltpu.SemaphoreType.DMA,
          pltpu.VMEM((cn + 16, 16), jnp.float32),
          pltpu.VMEM((16, 256), jnp.float32),
          pltpu.VMEM_SHARED((256, 256), jnp.float32),
          pltpu.VMEM_SHARED((16, REG), jnp.int32),
          pltpu.VMEM_SHARED((16, REG), jnp.int32),
          pltpu.VMEM_SHARED((16, 16), jnp.int32),
          pltpu.SemaphoreType.DMA,
      ],
  )
  def scat(dst_hbm, p_hbm, out_hbm,
           dwin, wids, wlds, cbuf, cvm,
           idbA, ldbA, ixbA, gbA, semgA, semlA,
           idbB, ldbB, ixbB, gbB, semgB, semlB,
           acc2, obuf, stg, ids_sp, lds_sp, cnt_sp, sem):
    bufA = (idbA, ldbA, ixbA, gbA, semgA, semlA)
    bufB = (idbB, ldbB, ixbB, gbB, semgB, semlB)
    c = lax.axis_index("c")
    s = lax.axis_index("s")
    iota = lax.iota(jnp.int32, 16)
    zero16 = jnp.zeros((16,), jnp.float32)
    zero16i = jnp.zeros((16,), jnp.int32)
    trash16 = jnp.full((16,), cn, jnp.int32)

    def one_pass(p, pcarry):
      base = pl.multiple_of(c * half + p * cn, 8)

      def az(j, carry):
        for q in range(8):
          acc2[j * 8 + q, :] = zero16
        return carry

      lax.fori_loop(0, (cn + 16) // 8, az, 0)

      def wloop(w, cntv):
        pltpu.sync_copy(dst_hbm.at[pl.ds(s * share + w * WIN, WIN)], dwin)

        def vec(v, wcv):
          d = dwin[pl.ds(v * 16, 16)]
          rel = d - base
          mask = (rel >= 0) & (rel < cn)
          pos = wcv - 1 + plsc.cumsum(mask.astype(jnp.int32))
          gid = s * share + w * WIN + v * 16 + iota
          plsc.store_scatter(wids, [pos], gid, mask=mask)
          plsc.store_scatter(wlds, [pos], rel, mask=mask)
          return wcv + plsc.all_reduce_population_count(mask)

        wcv = lax.fori_loop(0, WIN // 16, vec, jnp.zeros((16,), jnp.int32))
        wc = wcv[0]
        wids[pl.ds(wc, 16)] = zero16i
        wlds[pl.ds(wc, 16)] = trash16
        cnt = pl.multiple_of(cntv[0], 8)
        pltpu.sync_copy(wids, ids_sp.at[s, pl.ds(cnt, WIN + 16)])
        pltpu.sync_copy(wlds, lds_sp.at[s, pl.ds(cnt, WIN + 16)])
        return cntv + ((wcv + 7) & (-8))

      cntv = lax.fori_loop(0, nwin, wloop, jnp.zeros((16,), jnp.int32))
      cnt = pl.multiple_of(cntv[0], 8)
      for t in range(8):
        wids[pl.ds(t * 16, 16)] = zero16i
        wlds[pl.ds(t * 16, 16)] = trash16
      pltpu.sync_copy(wids.at[pl.ds(0, 128)], ids_sp.at[s, pl.ds(cnt, 128)])
      pltpu.sync_copy(wlds.at[pl.ds(0, 128)], lds_sp.at[s, pl.ds(cnt, 128)])
      cbuf[pl.ds(0, 16)] = cntv
      pltpu.sync_copy(cbuf, cnt_sp.at[s])
      plsc.subcore_barrier()

      pltpu.sync_copy(cnt_sp, cvm)

      def fetch(t, g, buf):
        idb, ldb, ixb, gb, semg, seml = buf
        pltpu.sync_copy(ids_sp.at[t, pl.ds(g * 128, 128)],
                        idb.at[pl.ds(0, 128)])
        pltpu.async_copy(lds_sp.at[t, pl.ds(g * 128, 128)],
                         ldb.at[pl.ds(0, 128)], seml)
        for q in range(8):
          ixb[pl.ds(q * 16, 16)] = idb[pl.ds(q * 16, 16)] * 16 + s
        pltpu.async_copy(p_hbm.at[ixb], gb, semg)

      def drain_acc(buf):
        idb, ldb, ixb, gb, semg, seml = buf
        pltpu.make_async_copy(lds_sp.at[0, pl.ds(0, 128)],
                              ldb.at[pl.ds(0, 128)], seml).wait()
        pltpu.make_async_copy(p_hbm.at[ixb], gb, semg).wait()

        def step(i, c2):
          rv = ldb[pl.ds(i, 16)]
          plsc.addupdate_scatter(acc2, [zero16i + rv[0], iota], gb[i, :])
          return c2

        lax.fori_loop(0, 128, step, 0)

      def walk_t(t, tcarry):
        cnt_t = cvm[t, :][0]
        nb = lax.div(cnt_t + 127, 128)

        @pl.when(nb > 0)
        def _():
          fetch(t, 0, bufA)

        def pairq(q, carry):
          g0 = 2 * q

          @pl.when(g0 + 1 < nb)
          def _():
            fetch(t, g0 + 1, bufB)

          drain_acc(bufA)

          @pl.when(g0 + 2 < nb)
          def _():
            fetch(t, g0 + 2, bufA)

          @pl.when(g0 + 1 < nb)
          def _():
            drain_acc(bufB)

          return carry

        lax.fori_loop(0, lax.div(nb + 1, 2), pairq, 0)
        return tcarry

      lax.fori_loop(0, NS, walk_t, 0)
      plsc.subcore_barrier()

      SB = 256
      rpt = SB // NS

      def unint(sb, carry):
        pltpu.sync_copy(acc2.at[pl.ds(pl.multiple_of(sb * SB, 8), SB)],
                        stg.at[:, pl.ds(s * 16, 16)])
        plsc.subcore_barrier()
        r0 = s * rpt
        pltpu.sync_copy(stg.at[pl.ds(r0, rpt)], obuf)
        ofs = pl.multiple_of(base + sb * SB + r0, 8)
        pltpu.sync_copy(obuf, out_hbm.at[pl.ds(ofs, rpt)])
        plsc.subcore_barrier()
        return carry

      lax.fori_loop(0, cn // SB, unint, 0)
      return pcarry

    lax.fori_loop(0, npass, one_pass, 0)

  return scat


def _egc_layer(h, ef, src, dst, p, ep, nout, half, cn, br_n, br_e):
  n = h.shape[0]
  e = ef.shape[0]

  wt = jnp.concatenate(
      [p['W_src_gate'].T, p['W_dst_update'].T,
       p['W_dst_gate'].T, p['W_src_update'].T], axis=1)
  bcat = jnp.concatenate(
      [p['b_src_gate'], p['b_dst_update'],
       p['b_dst_gate'], p['b_src_update']])
  b8 = jnp.tile(bcat[None, :], (8, 1))
  s_tab, bt_tab, c_tab = _mm3(h, wt, b8, br_n)

  wet = p['W_edge_gate'].T
  be8 = jnp.tile(p['b_edge_gate'][None, :], (8, 1))
  elin = _mm1_padded(ef, wet, be8, br_e, ep)

  npad = ep - e
  fill = jnp.arange(npad, dtype=jnp.int32) % n
  src_g = jnp.concatenate([src, fill])
  dst_g = jnp.concatenate([dst, fill])
  dst_s = jnp.concatenate([dst, jnp.full((npad,), 1 << 20, jnp.int32)])

  m1, pay = _make_gate(ep)(s_tab, bt_tab, elin, src_g, dst_g)
  ss = _make_scatter(ep, nout, half, cn)(dst_s, pay)

  g_n8 = jnp.tile(p['g_nodes'][None, :], (8, 1))
  b_n8 = jnp.tile(p['b_nodes'][None, :], (8, 1))
  g_e8 = jnp.tile(p['g_edges'][None, :], (8, 1))
  b_e8 = jnp.tile(p['b_edges'][None, :], (8, 1))

  x_out = _post_node(h, c_tab, ss, g_n8, b_n8, br_n)
  y_out = _post_edge(ef, m1, g_e8, b_e8, br_e)
  return x_out, y_out


def kernel(x, y, z, nu_params, eu_params, edge_index, lg_edge_index):
  src, dst = edge_index[0], edge_index[1]
  lsrc, ldst = lg_edge_index[0], lg_edge_index[1]

  x_out, m = _egc_layer(
      x, y, src, dst, nu_params,
      ep=163840, nout=10240, half=5120, cn=5120,
      br_n=400, br_e=640)

  y_out, z_out = _egc_layer(
      m, z, lsrc, ldst, eu_params,
      ep=327680, nout=163840, half=81920, cn=4096,
      br_n=640, br_e=640)

  return (x_out, y_out, z_out)
